# baseline (speedup 1.0000x reference)
import functools

import jax
import jax.numpy as jnp
from jax import lax
from jax.experimental import pallas as pl
from jax.experimental.pallas import tpu as pltpu
from jax.experimental.pallas import tpu_sc as plsc


NC, NS, LANES = 2, 16, 16
NW = NC * NS
OFF_WK, OFF_ST, OFF_WM = 0, 21, 336
TBL_PAD = 384


def _sc_traj_body(wd_hbm, st_hbm, semO_hbm, tbl_hbm, traj_hbm,
                  wd_s, st_s, semO_s, tbl_s, traj_s, dsem):
    per_w = wd_hbm.shape[0] // NW
    ngrp = per_w // LANES
    wid = lax.axis_index("s") * NC + lax.axis_index("c")
    base = wid * per_w
    iota = lax.iota(jnp.int32, LANES)

    descs = [
        pltpu.async_copy(tbl_hbm, tbl_s, dsem),
        pltpu.async_copy(wd_hbm.at[pl.ds(base, per_w)], wd_s, dsem),
        pltpu.async_copy(st_hbm.at[pl.ds(base, per_w)], st_s, dsem),
        pltpu.async_copy(semO_hbm.at[pl.ds(base * 8, per_w * 8)], semO_s,
                         dsem),
    ]
    for d_ in descs:
        d_.wait()
    wsp = [[plsc.load_gather(tbl_s, [jnp.full((LANES,), OFF_WM + d * 8 + k,
                                              jnp.int32)])
            for k in range(8)] for d in range(3)]
    for g in range(ngrp):
        wd16 = wd_s[pl.ds(g * LANES, LANES)]
        st16 = st_s[pl.ds(g * LANES, LANES)]
        comps = []
        for d in range(3):
            comps.append(plsc.load_gather(tbl_s, [wd16 * 3 + (OFF_WK + d)]))
        for d in range(6):
            comps.append(plsc.load_gather(tbl_s, [st16 * 6 + (OFF_ST + d)]))
        cols = [plsc.load_gather(semO_s, [(g * LANES + iota) * 8 + k])
                for k in range(8)]
        for d in range(3):
            acc = cols[0] * wsp[d][0]
            for k in range(1, 8):
                acc = acc + cols[k] * wsp[d][k]
            comps.append(acc)
        for j in range(12):
            plsc.store_scatter(traj_s, [(g * LANES + iota) * 12 + j],
                               comps[j])
    plsc.subcore_barrier()
    pltpu.sync_copy(traj_s, traj_hbm.at[pl.ds(base * 12, per_w * 12)])


def _sc_traj(weekday, start_time, sem_O, weekday_table, start_time_table,
             W_map):
    B = weekday.shape[0]
    f32, i32 = jnp.float32, jnp.int32
    per_w = B // NW
    tbl = jnp.concatenate([
        weekday_table.reshape(-1), start_time_table.reshape(-1),
        jnp.zeros((OFF_WM - 309,), f32),
        W_map.reshape(-1), jnp.zeros((TBL_PAD - 360,), f32),
    ])
    call = pl.kernel(
        _sc_traj_body,
        out_type=jax.ShapeDtypeStruct((B * 12,), f32),
        mesh=plsc.VectorSubcoreMesh(core_axis_name="c", subcore_axis_name="s",
                                    num_cores=NC, num_subcores=NS),
        compiler_params=pltpu.CompilerParams(needs_layout_passes=False),
        scratch_types=[
            pltpu.VMEM((per_w,), i32), pltpu.VMEM((per_w,), i32),
            pltpu.VMEM((per_w * 8,), f32), pltpu.VMEM((TBL_PAD,), f32),
            pltpu.VMEM((per_w * 12,), f32), pltpu.SemaphoreType.DMA,
        ],
    )
    traj_flat = call(weekday.astype(i32), start_time.astype(i32),
                     sem_O.reshape(-1), tbl)
    return traj_flat.reshape(B, 12)


def _tc_body(L, wd_ref, st_ref, semO_ref, lngs_ref, lats_ref, dis_ref,
             spd_ref, azi_ref, spt_ref, Wbig_ref, tbl_ref, out_ref):
    TB = wd_ref.shape[0]
    lane = jax.lax.broadcasted_iota(jnp.int32, (TB, 55), 1)
    target = jnp.where(lane < 7, wd_ref[:, :], st_ref[:, :] + 7)
    oh = (lane == target).astype(jnp.float32)
    feats = jnp.concatenate([oh, semO_ref[:, :]], axis=1)
    traj = jnp.dot(feats, Wbig_ref[:, :], preferred_element_type=jnp.float32)

    spt = spt_ref[:, :]
    embs = []
    for d in range(3):
        acc = jnp.zeros((TB, L), jnp.float32)
        for k in range(1, 9):
            acc = acc + jnp.where(spt == k, tbl_ref[k, d], 0.0)
        embs.append(acc)

    chans8 = [lngs_ref[:, :], lats_ref[:, :], dis_ref[:, :], spd_ref[:, :],
              azi_ref[:, :]] + embs
    j = 0
    for LC in (128, 64, 8):
        stk = jnp.stack([c[:, j:j + LC] for c in chans8], axis=1)
        t = jnp.swapaxes(stk, 1, 2)
        trajblk = jnp.broadcast_to(traj[:, None, :], (TB, LC, 12))
        out_ref[:, j:j + LC, :] = jnp.concatenate(
            [t[:, :, 0:5], trajblk, t[:, :, 5:8]], axis=2)
        j += LC


def _tc_input_tensor(weekday, start_time, sem_O, lngs, lats, sem_pt,
                     travel_dis, spd, azimuth, weekday_table,
                     start_time_table, sem_pt_table, W_map):
    B, L = lngs.shape
    TB = 128
    Wbig = jnp.zeros((63, 12), jnp.float32)
    Wbig = Wbig.at[0:7, 0:3].set(weekday_table)
    Wbig = Wbig.at[7:55, 3:9].set(start_time_table)
    Wbig = Wbig.at[55:63, 9:12].set(W_map.T)
    wd2 = weekday.astype(jnp.int32).reshape(B, 1)
    st2 = start_time.astype(jnp.int32).reshape(B, 1)

    out = pl.pallas_call(
        functools.partial(_tc_body, L),
        grid=(B // TB,),
        in_specs=[
            pl.BlockSpec((TB, 1), lambda i: (i, 0)),
            pl.BlockSpec((TB, 1), lambda i: (i, 0)),
            pl.BlockSpec((TB, 8), lambda i: (i, 0)),
            pl.BlockSpec((TB, L), lambda i: (i, 0)),
            pl.BlockSpec((TB, L), lambda i: (i, 0)),
            pl.BlockSpec((TB, L), lambda i: (i, 0)),
            pl.BlockSpec((TB, L), lambda i: (i, 0)),
            pl.BlockSpec((TB, L), lambda i: (i, 0)),
            pl.BlockSpec((TB, L), lambda i: (i, 0)),
            pl.BlockSpec((63, 12), lambda i: (0, 0)),
            pl.BlockSpec((9, 3), lambda i: (0, 0)),
        ],
        out_specs=pl.BlockSpec((TB, L, 20), lambda i: (i, 0, 0)),
        out_shape=jax.ShapeDtypeStruct((B, L, 20), jnp.float32),
    )(wd2, st2, sem_O, lngs, lats, travel_dis, spd, azimuth,
      sem_pt.astype(jnp.int32), Wbig, sem_pt_table)
    return out


def kernel(weekday, start_time, sem_O, lngs, lats, sem_pt, travel_dis, spd,
           azimuth, weekday_table, start_time_table, sem_pt_table, W_map):
    traj = _sc_traj(weekday, start_time, sem_O, weekday_table,
                    start_time_table, W_map)
    out = _tc_input_tensor(weekday, start_time, sem_O, lngs, lats, sem_pt,
                           travel_dis, spd, azimuth, weekday_table,
                           start_time_table, sem_pt_table, W_map)
    return out, traj


# speedup vs baseline: 1.0005x; 1.0005x over previous
import functools

import jax
import jax.numpy as jnp
from jax import lax
from jax.experimental import pallas as pl
from jax.experimental.pallas import tpu as pltpu
from jax.experimental.pallas import tpu_sc as plsc


NC, NS, LANES = 2, 16, 16
NW = NC * NS
OFF_WK, OFF_ST, OFF_WM = 0, 21, 336
TBL_PAD = 384


def _sc_traj_body(wd_hbm, st_hbm, semO_hbm, tbl_hbm, traj_hbm,
                  wd_s, st_s, semO_s, tbl_s, traj_s, dsem):
    per_w = wd_hbm.shape[0] // NW
    ngrp = per_w // LANES
    wid = lax.axis_index("s") * NC + lax.axis_index("c")
    base = wid * per_w
    iota = lax.iota(jnp.int32, LANES)

    descs = [
        pltpu.async_copy(tbl_hbm, tbl_s, dsem),
        pltpu.async_copy(wd_hbm.at[pl.ds(base, per_w)], wd_s, dsem),
        pltpu.async_copy(st_hbm.at[pl.ds(base, per_w)], st_s, dsem),
        pltpu.async_copy(semO_hbm.at[pl.ds(base * 8, per_w * 8)], semO_s,
                         dsem),
    ]
    for d_ in descs:
        d_.wait()
    wsp = [[plsc.load_gather(tbl_s, [jnp.full((LANES,), OFF_WM + d * 8 + k,
                                              jnp.int32)])
            for k in range(8)] for d in range(3)]
    for g in range(ngrp):
        wd16 = wd_s[pl.ds(g * LANES, LANES)]
        st16 = st_s[pl.ds(g * LANES, LANES)]
        comps = []
        for d in range(3):
            comps.append(plsc.load_gather(tbl_s, [wd16 * 3 + (OFF_WK + d)]))
        for d in range(6):
            comps.append(plsc.load_gather(tbl_s, [st16 * 6 + (OFF_ST + d)]))
        cols = [plsc.load_gather(semO_s, [(g * LANES + iota) * 8 + k])
                for k in range(8)]
        for d in range(3):
            acc = cols[0] * wsp[d][0]
            for k in range(1, 8):
                acc = acc + cols[k] * wsp[d][k]
            comps.append(acc)
        for j in range(12):
            plsc.store_scatter(traj_s, [(g * LANES + iota) * 12 + j],
                               comps[j])
    plsc.subcore_barrier()
    pltpu.sync_copy(traj_s, traj_hbm.at[pl.ds(base * 12, per_w * 12)])


def _sc_traj(weekday, start_time, sem_O, weekday_table, start_time_table,
             W_map):
    B = weekday.shape[0]
    f32, i32 = jnp.float32, jnp.int32
    per_w = B // NW
    tbl = jnp.concatenate([
        weekday_table.reshape(-1), start_time_table.reshape(-1),
        jnp.zeros((OFF_WM - 309,), f32),
        W_map.reshape(-1), jnp.zeros((TBL_PAD - 360,), f32),
    ])
    call = pl.kernel(
        _sc_traj_body,
        out_type=jax.ShapeDtypeStruct((B * 12,), f32),
        mesh=plsc.VectorSubcoreMesh(core_axis_name="c", subcore_axis_name="s",
                                    num_cores=NC, num_subcores=NS),
        compiler_params=pltpu.CompilerParams(needs_layout_passes=False),
        scratch_types=[
            pltpu.VMEM((per_w,), i32), pltpu.VMEM((per_w,), i32),
            pltpu.VMEM((per_w * 8,), f32), pltpu.VMEM((TBL_PAD,), f32),
            pltpu.VMEM((per_w * 12,), f32), pltpu.SemaphoreType.DMA,
        ],
    )
    traj_flat = call(weekday.astype(i32), start_time.astype(i32),
                     sem_O.reshape(-1), tbl)
    return traj_flat.reshape(B, 12)


def _tc_body(L, wd_ref, st_ref, semO_ref, lngs_ref, lats_ref, dis_ref,
             spd_ref, azi_ref, spt_ref, Wbig_ref, tbl_ref, out_ref):
    TB = wd_ref.shape[0]
    lane = jax.lax.broadcasted_iota(jnp.int32, (TB, 55), 1)
    target = jnp.where(lane < 7, wd_ref[:, :], st_ref[:, :] + 7)
    oh = (lane == target).astype(jnp.float32)
    feats = jnp.concatenate([oh, semO_ref[:, :]], axis=1)
    traj = jnp.dot(feats, Wbig_ref[:, :], preferred_element_type=jnp.float32)

    spt = spt_ref[:, :]
    embs = []
    for d in range(3):
        acc = jnp.zeros((TB, L), jnp.float32)
        for k in range(1, 9):
            acc = acc + jnp.where(spt == k, tbl_ref[k, d], 0.0)
        embs.append(acc)

    chans8 = [lngs_ref[:, :], lats_ref[:, :], dis_ref[:, :], spd_ref[:, :],
              azi_ref[:, :]] + embs
    j = 0
    for LC in (128, 64, 8):
        stk = jnp.stack([c[:, j:j + LC] for c in chans8], axis=1)
        t = jnp.swapaxes(stk, 1, 2)
        trajblk = jnp.broadcast_to(traj[:, None, :], (TB, LC, 12))
        out_ref[:, j:j + LC, :] = jnp.concatenate(
            [t[:, :, 0:5], trajblk, t[:, :, 5:8]], axis=2)
        j += LC


def _tc_input_tensor(weekday, start_time, sem_O, lngs, lats, sem_pt,
                     travel_dis, spd, azimuth, weekday_table,
                     start_time_table, sem_pt_table, W_map):
    B, L = lngs.shape
    TB = 128
    Wbig = jnp.zeros((63, 12), jnp.float32)
    Wbig = Wbig.at[0:7, 0:3].set(weekday_table)
    Wbig = Wbig.at[7:55, 3:9].set(start_time_table)
    Wbig = Wbig.at[55:63, 9:12].set(W_map.T)
    wd2 = weekday.astype(jnp.int32).reshape(B, 1)
    st2 = start_time.astype(jnp.int32).reshape(B, 1)

    out = pl.pallas_call(
        functools.partial(_tc_body, L),
        grid=(B // TB,),
        in_specs=[
            pl.BlockSpec((TB, 1), lambda i: (i, 0)),
            pl.BlockSpec((TB, 1), lambda i: (i, 0)),
            pl.BlockSpec((TB, 8), lambda i: (i, 0)),
            pl.BlockSpec((TB, L), lambda i: (i, 0)),
            pl.BlockSpec((TB, L), lambda i: (i, 0)),
            pl.BlockSpec((TB, L), lambda i: (i, 0)),
            pl.BlockSpec((TB, L), lambda i: (i, 0)),
            pl.BlockSpec((TB, L), lambda i: (i, 0)),
            pl.BlockSpec((TB, L), lambda i: (i, 0)),
            pl.BlockSpec((63, 12), lambda i: (0, 0)),
            pl.BlockSpec((9, 3), lambda i: (0, 0)),
        ],
        out_specs=pl.BlockSpec((TB, L, 20), lambda i: (i, 0, 0)),
        out_shape=jax.ShapeDtypeStruct((B, L, 20), jnp.float32),
        compiler_params=pltpu.CompilerParams(
            dimension_semantics=("parallel",)),
    )(wd2, st2, sem_O, lngs, lats, travel_dis, spd, azimuth,
      sem_pt.astype(jnp.int32), Wbig, sem_pt_table)
    return out


def kernel(weekday, start_time, sem_O, lngs, lats, sem_pt, travel_dis, spd,
           azimuth, weekday_table, start_time_table, sem_pt_table, W_map):
    traj = _sc_traj(weekday, start_time, sem_O, weekday_table,
                    start_time_table, W_map)
    out = _tc_input_tensor(weekday, start_time, sem_O, lngs, lats, sem_pt,
                           travel_dis, spd, azimuth, weekday_table,
                           start_time_table, sem_pt_table, W_map)
    return out, traj


# speedup vs baseline: 1.0118x; 1.0113x over previous
import functools

import jax
import jax.numpy as jnp
from jax import lax
from jax.experimental import pallas as pl
from jax.experimental.pallas import tpu as pltpu
from jax.experimental.pallas import tpu_sc as plsc


NC, NS, LANES = 2, 16, 16
NW = NC * NS
OFF_WK, OFF_ST, OFF_WM = 0, 21, 336
TBL_PAD = 384


def _sc_traj_body(wd_hbm, st_hbm, semO_hbm, tbl_hbm, traj_hbm,
                  wd_s, st_s, semO_s, tbl_s, traj_s, dsem):
    per_w = wd_hbm.shape[0] // NW
    ngrp = per_w // LANES
    wid = lax.axis_index("s") * NC + lax.axis_index("c")
    base = wid * per_w
    iota = lax.iota(jnp.int32, LANES)

    descs = [
        pltpu.async_copy(tbl_hbm, tbl_s, dsem),
        pltpu.async_copy(wd_hbm.at[pl.ds(base, per_w)], wd_s, dsem),
        pltpu.async_copy(st_hbm.at[pl.ds(base, per_w)], st_s, dsem),
        pltpu.async_copy(semO_hbm.at[pl.ds(base * 8, per_w * 8)], semO_s,
                         dsem),
    ]
    for d_ in descs:
        d_.wait()
    wsp = [[plsc.load_gather(tbl_s, [jnp.full((LANES,), OFF_WM + d * 8 + k,
                                              jnp.int32)])
            for k in range(8)] for d in range(3)]
    for g in range(ngrp):
        wd16 = wd_s[pl.ds(g * LANES, LANES)]
        st16 = st_s[pl.ds(g * LANES, LANES)]
        comps = []
        for d in range(3):
            comps.append(plsc.load_gather(tbl_s, [wd16 * 3 + (OFF_WK + d)]))
        for d in range(6):
            comps.append(plsc.load_gather(tbl_s, [st16 * 6 + (OFF_ST + d)]))
        cols = [plsc.load_gather(semO_s, [(g * LANES + iota) * 8 + k])
                for k in range(8)]
        for d in range(3):
            acc = cols[0] * wsp[d][0]
            for k in range(1, 8):
                acc = acc + cols[k] * wsp[d][k]
            comps.append(acc)
        for j in range(12):
            plsc.store_scatter(traj_s, [(g * LANES + iota) * 12 + j],
                               comps[j])
    plsc.subcore_barrier()
    pltpu.sync_copy(traj_s, traj_hbm.at[pl.ds(base * 12, per_w * 12)])


def _sc_traj(weekday, start_time, sem_O, weekday_table, start_time_table,
             W_map):
    B = weekday.shape[0]
    f32, i32 = jnp.float32, jnp.int32
    per_w = B // NW
    tbl = jnp.concatenate([
        weekday_table.reshape(-1), start_time_table.reshape(-1),
        jnp.zeros((OFF_WM - 309,), f32),
        W_map.reshape(-1), jnp.zeros((TBL_PAD - 360,), f32),
    ])
    call = pl.kernel(
        _sc_traj_body,
        out_type=jax.ShapeDtypeStruct((B * 12,), f32),
        mesh=plsc.VectorSubcoreMesh(core_axis_name="c", subcore_axis_name="s",
                                    num_cores=NC, num_subcores=NS),
        compiler_params=pltpu.CompilerParams(needs_layout_passes=False),
        scratch_types=[
            pltpu.VMEM((per_w,), i32), pltpu.VMEM((per_w,), i32),
            pltpu.VMEM((per_w * 8,), f32), pltpu.VMEM((TBL_PAD,), f32),
            pltpu.VMEM((per_w * 12,), f32), pltpu.SemaphoreType.DMA,
        ],
    )
    traj_flat = call(weekday.astype(i32), start_time.astype(i32),
                     sem_O.reshape(-1), tbl)
    return traj_flat.reshape(B, 12)


def _tc_body(L, wd_ref, st_ref, semO_ref, lngs_ref, lats_ref, dis_ref,
             spd_ref, azi_ref, spt_ref, Wbig_ref, tbl_ref, out_ref):
    TB = wd_ref.shape[0]
    lane = jax.lax.broadcasted_iota(jnp.int32, (TB, 55), 1)
    target = jnp.where(lane < 7, wd_ref[:, :], st_ref[:, :] + 7)
    oh = (lane == target).astype(jnp.float32)
    feats = jnp.concatenate([oh, semO_ref[:, :]], axis=1)
    traj = jnp.dot(feats, Wbig_ref[:, :], preferred_element_type=jnp.float32)

    spt = spt_ref[:, :]
    embs = []
    for d in range(3):
        acc = jnp.zeros((TB, L), jnp.float32)
        for k in range(1, 9):
            acc = acc + jnp.where(spt == k, tbl_ref[k, d], 0.0)
        embs.append(acc)

    chans8 = [lngs_ref[:, :], lats_ref[:, :], dis_ref[:, :], spd_ref[:, :],
              azi_ref[:, :]] + embs
    j = 0
    for LC in (128, 72):
        stk = jnp.stack([c[:, j:j + LC] for c in chans8], axis=1)
        t = jnp.swapaxes(stk, 1, 2)
        trajblk = jnp.broadcast_to(traj[:, None, :], (TB, LC, 12))
        out_ref[:, j:j + LC, :] = jnp.concatenate(
            [t[:, :, 0:5], trajblk, t[:, :, 5:8]], axis=2)
        j += LC


def _tc_input_tensor(weekday, start_time, sem_O, lngs, lats, sem_pt,
                     travel_dis, spd, azimuth, weekday_table,
                     start_time_table, sem_pt_table, W_map):
    B, L = lngs.shape
    TB = 128
    Wbig = jnp.zeros((63, 12), jnp.float32)
    Wbig = Wbig.at[0:7, 0:3].set(weekday_table)
    Wbig = Wbig.at[7:55, 3:9].set(start_time_table)
    Wbig = Wbig.at[55:63, 9:12].set(W_map.T)
    wd2 = weekday.astype(jnp.int32).reshape(B, 1)
    st2 = start_time.astype(jnp.int32).reshape(B, 1)

    out = pl.pallas_call(
        functools.partial(_tc_body, L),
        grid=(B // TB,),
        in_specs=[
            pl.BlockSpec((TB, 1), lambda i: (i, 0)),
            pl.BlockSpec((TB, 1), lambda i: (i, 0)),
            pl.BlockSpec((TB, 8), lambda i: (i, 0)),
            pl.BlockSpec((TB, L), lambda i: (i, 0)),
            pl.BlockSpec((TB, L), lambda i: (i, 0)),
            pl.BlockSpec((TB, L), lambda i: (i, 0)),
            pl.BlockSpec((TB, L), lambda i: (i, 0)),
            pl.BlockSpec((TB, L), lambda i: (i, 0)),
            pl.BlockSpec((TB, L), lambda i: (i, 0)),
            pl.BlockSpec((63, 12), lambda i: (0, 0)),
            pl.BlockSpec((9, 3), lambda i: (0, 0)),
        ],
        out_specs=pl.BlockSpec((TB, L, 20), lambda i: (i, 0, 0)),
        out_shape=jax.ShapeDtypeStruct((B, L, 20), jnp.float32),
        compiler_params=pltpu.CompilerParams(
            dimension_semantics=("parallel",)),
    )(wd2, st2, sem_O, lngs, lats, travel_dis, spd, azimuth,
      sem_pt.astype(jnp.int32), Wbig, sem_pt_table)
    return out


def kernel(weekday, start_time, sem_O, lngs, lats, sem_pt, travel_dis, spd,
           azimuth, weekday_table, start_time_table, sem_pt_table, W_map):
    traj = _sc_traj(weekday, start_time, sem_O, weekday_table,
                    start_time_table, W_map)
    out = _tc_input_tensor(weekday, start_time, sem_O, lngs, lats, sem_pt,
                           travel_dis, spd, azimuth, weekday_table,
                           start_time_table, sem_pt_table, W_map)
    return out, traj
